# baseline (speedup 1.0000x reference)
import functools

import numpy as np
import jax
import jax.numpy as jnp
from jax import lax
from jax.experimental import pallas as pl
from jax.experimental.pallas import tpu as pltpu
from jax.experimental.pallas import tpu_sc as plsc

_N_BINS = 10
_LANES = 16
_NW = 32
_CHUNK = 1024
_HSLOTS = _N_BINS + 1
_HSIZE = 3 * _HSLOTS * _LANES

_BOUNDS = [float(x) for x in np.linspace(0.0, 1.0, _N_BINS + 1).astype(np.float32)[:-1]]


@functools.cache
def _make_sc_hist(n_rows: int, n_cols: int):
    rows_per_tile = n_rows // _NW
    n_chunks = rows_per_tile // _CHUNK
    n_pairs = n_chunks // 2
    assert n_rows == _NW * rows_per_tile
    assert rows_per_tile == n_chunks * _CHUNK and n_chunks % 2 == 0
    groups_per_chunk = _CHUNK // _LANES

    mesh = plsc.VectorSubcoreMesh(core_axis_name="c", subcore_axis_name="s")

    @functools.partial(
        pl.kernel,
        mesh=mesh,
        compiler_params=pltpu.CompilerParams(needs_layout_passes=False),
        out_type=jax.ShapeDtypeStruct((_NW, _HSIZE), jnp.float32),
        scratch_types=[
            pltpu.VMEM((_CHUNK * n_cols,), jnp.float32),
            pltpu.VMEM((_CHUNK * n_cols,), jnp.float32),
            pltpu.VMEM((_CHUNK,), jnp.int32),
            pltpu.VMEM((_CHUNK,), jnp.int32),
            pltpu.VMEM((_HSIZE,), jnp.float32),
            pltpu.SemaphoreType.DMA,
            pltpu.SemaphoreType.DMA,
            pltpu.SemaphoreType.DMA,
            pltpu.SemaphoreType.DMA,
        ],
    )
    def hist_kernel(logits_hbm, labels_hbm, out_hbm,
                    lbuf0, lbuf1, labbuf0, labbuf1, hist,
                    sem0, sem1, lsem0, lsem1):
        wid = lax.axis_index("s") * 2 + lax.axis_index("c")
        base_row = wid * rows_per_tile

        iota = lax.iota(jnp.int32, 16)
        iota_c = iota * n_cols
        zeros16 = jnp.zeros((16,), jnp.float32)
        ones16 = jnp.ones((16,), jnp.float32)

        for i in range(_HSIZE // 16):
            hist[pl.ds(i * 16, 16)] = zeros16

        def start_chunk(ci, buf, labbuf, s_l, s_lab):
            r0 = base_row + ci * _CHUNK
            pltpu.async_copy(
                logits_hbm.at[pl.ds(r0 * n_cols, _CHUNK * n_cols)], buf, s_l)
            pltpu.async_copy(labels_hbm.at[pl.ds(r0, _CHUNK)], labbuf, s_lab)

        def wait_chunk(ci, buf, labbuf, s_l, s_lab):
            r0 = base_row + ci * _CHUNK
            pltpu.make_async_copy(
                logits_hbm.at[pl.ds(r0 * n_cols, _CHUNK * n_cols)], buf, s_l).wait()
            pltpu.make_async_copy(
                labels_hbm.at[pl.ds(r0, _CHUNK)], labbuf, s_lab).wait()

        def process(buf, labbuf):
            @plsc.parallel_loop(0, groups_per_chunk, unroll=2)
            def group_body(g):
                base = iota_c + g * (_LANES * n_cols)
                vs = [plsc.load_gather(buf, [base + c]) for c in range(n_cols)]
                while len(vs) > 1:
                    nxt = [jnp.maximum(vs[2 * i], vs[2 * i + 1])
                           for i in range(len(vs) // 2)]
                    if len(vs) % 2:
                        nxt.append(vs[-1])
                    vs = nxt
                m = vs[0]
                lab = labbuf[pl.ds(g * _LANES, _LANES)]
                vlab = plsc.load_gather(buf, [base + lab])
                acc = jnp.where(vlab == m, ones16, zeros16)
                binv = jnp.minimum((m * 10.0).astype(jnp.int32), 9)
                binv = jnp.where(m > 0.0, binv, _N_BINS)
                slot = binv * _LANES + iota
                plsc.addupdate_scatter(hist, [slot], ones16)
                plsc.addupdate_scatter(hist, [slot + _HSLOTS * _LANES], acc)
                plsc.addupdate_scatter(hist, [slot + 2 * _HSLOTS * _LANES], m)

        start_chunk(0, lbuf0, labbuf0, sem0, lsem0)

        def pair_body(p, carry):
            c0 = 2 * p
            start_chunk(c0 + 1, lbuf1, labbuf1, sem1, lsem1)
            wait_chunk(c0, lbuf0, labbuf0, sem0, lsem0)
            process(lbuf0, labbuf0)

            @pl.when(p < n_pairs - 1)
            def _():
                start_chunk(c0 + 2, lbuf0, labbuf0, sem0, lsem0)

            wait_chunk(c0 + 1, lbuf1, labbuf1, sem1, lsem1)
            process(lbuf1, labbuf1)
            return carry

        lax.fori_loop(0, n_pairs, pair_body, 0)

        pltpu.sync_copy(hist, out_hbm.at[wid])

    return hist_kernel


def kernel(logits, labels):
    n_rows, n_cols = logits.shape
    partials = _make_sc_hist(n_rows, n_cols)(
        logits.reshape(-1), labels.astype(jnp.int32))
    h = partials.sum(axis=0).reshape(3, _HSLOTS, _LANES).sum(axis=-1)
    cnt = h[0, :_N_BINS]
    accs = h[1, :_N_BINS]
    confs = h[2, :_N_BINS]
    prop = cnt / n_rows
    safe = jnp.maximum(cnt, 1.0)
    contrib = jnp.abs(confs / safe - accs / safe) * prop
    ece = jnp.sum(jnp.where(prop > 0, contrib, 0.0))
    return ece.reshape(1).astype(logits.dtype)


# speedup vs baseline: 1.7233x; 1.7233x over previous
import functools

import numpy as np
import jax
import jax.numpy as jnp
from jax import lax
from jax.experimental import pallas as pl
from jax.experimental.pallas import tpu as pltpu
from jax.experimental.pallas import tpu_sc as plsc

_N_BINS = 10
_LANES = 16
_NW = 32
_CHUNK = 1024
_HSLOTS = _N_BINS + 1
_HSIZE = 3 * _HSLOTS * _LANES

_BOUNDS = [float(x) for x in np.linspace(0.0, 1.0, _N_BINS + 1).astype(np.float32)[:-1]]


@functools.cache
def _make_sc_hist(n_rows: int, n_cols: int):
    rows_per_tile = n_rows // _NW
    n_chunks = rows_per_tile // _CHUNK
    n_pairs = n_chunks // 2
    assert n_rows == _NW * rows_per_tile
    assert rows_per_tile == n_chunks * _CHUNK and n_chunks % 2 == 0
    groups_per_chunk = _CHUNK // _LANES

    mesh = plsc.VectorSubcoreMesh(core_axis_name="c", subcore_axis_name="s")

    @functools.partial(
        pl.kernel,
        mesh=mesh,
        compiler_params=pltpu.CompilerParams(needs_layout_passes=False),
        out_type=jax.ShapeDtypeStruct((_NW, _HSIZE), jnp.float32),
        scratch_types=[
            pltpu.VMEM((_CHUNK * n_cols,), jnp.float32),
            pltpu.VMEM((_CHUNK * n_cols,), jnp.float32),
            pltpu.VMEM((_CHUNK,), jnp.int32),
            pltpu.VMEM((_CHUNK,), jnp.int32),
            pltpu.VMEM((_HSIZE,), jnp.float32),
            pltpu.SemaphoreType.DMA,
            pltpu.SemaphoreType.DMA,
            pltpu.SemaphoreType.DMA,
            pltpu.SemaphoreType.DMA,
        ],
    )
    def hist_kernel(logits_hbm, labels_hbm, out_hbm,
                    lbuf0, lbuf1, labbuf0, labbuf1, hist,
                    sem0, sem1, lsem0, lsem1):
        wid = lax.axis_index("s") * 2 + lax.axis_index("c")
        base_row = wid * rows_per_tile

        iota = lax.iota(jnp.int32, 16)
        iota_c = iota * n_cols
        zeros16 = jnp.zeros((16,), jnp.float32)
        ones16 = jnp.ones((16,), jnp.float32)

        for i in range(_HSIZE // 16):
            hist[pl.ds(i * 16, 16)] = zeros16

        def start_chunk(ci, buf, labbuf, s_l, s_lab):
            r0 = base_row + ci * _CHUNK
            pltpu.async_copy(
                logits_hbm.at[pl.ds(r0 * n_cols, _CHUNK * n_cols)], buf, s_l)
            pltpu.async_copy(labels_hbm.at[pl.ds(r0, _CHUNK)], labbuf, s_lab)

        def wait_chunk(ci, buf, labbuf, s_l, s_lab):
            r0 = base_row + ci * _CHUNK
            pltpu.make_async_copy(
                logits_hbm.at[pl.ds(r0 * n_cols, _CHUNK * n_cols)], buf, s_l).wait()
            pltpu.make_async_copy(
                labels_hbm.at[pl.ds(r0, _CHUNK)], labbuf, s_lab).wait()

        def process(buf, labbuf):
            @plsc.parallel_loop(0, groups_per_chunk, unroll=2)
            def group_body(g):
                base = iota_c + g * (_LANES * n_cols)
                vs = []
                ck = iota
                for k in range(n_cols):
                    vs.append(plsc.load_gather(buf, [base + ck]))
                    if k < n_cols - 1:
                        ck = (ck + 1) & (n_cols - 1)
                while len(vs) > 1:
                    nxt = [jnp.maximum(vs[2 * i], vs[2 * i + 1])
                           for i in range(len(vs) // 2)]
                    if len(vs) % 2:
                        nxt.append(vs[-1])
                    vs = nxt
                m = vs[0]
                lab = labbuf[pl.ds(g * _LANES, _LANES)]
                vlab = plsc.load_gather(buf, [base + lab])
                acc = jnp.where(vlab == m, ones16, zeros16)
                binv = jnp.minimum((m * 10.0).astype(jnp.int32), 9)
                binv = jnp.where(m > 0.0, binv, _N_BINS)
                slot = binv * _LANES + iota
                plsc.addupdate_scatter(hist, [slot], ones16)
                plsc.addupdate_scatter(hist, [slot + _HSLOTS * _LANES], acc)
                plsc.addupdate_scatter(hist, [slot + 2 * _HSLOTS * _LANES], m)

        start_chunk(0, lbuf0, labbuf0, sem0, lsem0)

        def pair_body(p, carry):
            c0 = 2 * p
            start_chunk(c0 + 1, lbuf1, labbuf1, sem1, lsem1)
            wait_chunk(c0, lbuf0, labbuf0, sem0, lsem0)
            process(lbuf0, labbuf0)

            @pl.when(p < n_pairs - 1)
            def _():
                start_chunk(c0 + 2, lbuf0, labbuf0, sem0, lsem0)

            wait_chunk(c0 + 1, lbuf1, labbuf1, sem1, lsem1)
            process(lbuf1, labbuf1)
            return carry

        lax.fori_loop(0, n_pairs, pair_body, 0)

        pltpu.sync_copy(hist, out_hbm.at[wid])

    return hist_kernel


def kernel(logits, labels):
    n_rows, n_cols = logits.shape
    partials = _make_sc_hist(n_rows, n_cols)(
        logits.reshape(-1), labels.astype(jnp.int32))
    h = partials.sum(axis=0).reshape(3, _HSLOTS, _LANES).sum(axis=-1)
    cnt = h[0, :_N_BINS]
    accs = h[1, :_N_BINS]
    confs = h[2, :_N_BINS]
    prop = cnt / n_rows
    safe = jnp.maximum(cnt, 1.0)
    contrib = jnp.abs(confs / safe - accs / safe) * prop
    ece = jnp.sum(jnp.where(prop > 0, contrib, 0.0))
    return ece.reshape(1).astype(logits.dtype)
